# Initial kernel scaffold; baseline (speedup 1.0000x reference)
import jax
import jax.numpy as jnp
from jax.experimental import pallas as pl


def kernel(z, edge_index, batch, use_feature, embedding, z_table, W0, b0, W1, b1, W2, b2, W3, b3, conv1_w, conv1_b, conv2_w, conv2_b, lin1_w, lin1_b, lin2_w, lin2_b):
    raise NotImplementedError("write your pallas kernel here")


# speedup vs baseline: 16.7209x; 16.7209x over previous
import functools

import jax
import jax.numpy as jnp
from jax import lax
from jax.experimental import pallas as pl
from jax.experimental.pallas import tpu as pltpu
from jax.experimental.pallas import tpu_sc as plsc

N = 100000
E = 1600000
G = 128
H = 32
HH = 16
MAX_Z = 1000
K = 30
TOTAL_LATENT = H * 3 + 1
DENSE_DIM = 352

NTILES = 16
NCORES = 2
CH = 128
NC = 784
TE = NC * CH
E_TOT = NTILES * TE
NCW = E_TOT // (32 * CH)
NP = 100096
NROW_T = NP // NTILES
NSP = 100352
NODE_NC = NSP // NTILES // CH
C = 2048
GP = 130
KEYSLEN = GP * C
SELW = 32
SEL = G * SELW

NEG = -3.0e38
f32 = jnp.float32
i32 = jnp.int32

@functools.cache
def _mesh():
    return plsc.VectorSubcoreMesh(core_axis_name="c", subcore_axis_name="s")


def _fill_vec(ref, n, value, dtype):
    v = jnp.full((16,), value, dtype)
    for j in range(n // 16):
        ref[pl.ds(j * 16, 16)] = v


def _sc_deg_body(dst_hbm, batch_hbm, z_hbm, t0a_hbm, t0b_hbm,
                 dega_hbm, degb_hbm, counts_hbm, xga_hbm, xgb_hbm,
                 deg_sh, cnt_sh, ones_v, idx_v, zidx_v, rows_v, obuf_v,
                 sem0, sem1):
    cid = lax.axis_index("c")
    sid = lax.axis_index("s")

    _fill_vec(obuf_v, 1024, 0.0, f32)
    base0 = sid * NROW_T
    for i in range(0, NROW_T, 1024):
        sz = min(1024, NROW_T - i)
        pltpu.sync_copy(obuf_v.at[pl.ds(0, sz)], deg_sh.at[pl.ds(base0 + i, sz)])

    @pl.when(sid == 0)
    def _():
        pltpu.sync_copy(obuf_v.at[pl.ds(0, 144)], cnt_sh)

    _fill_vec(ones_v, CH, 1.0, f32)
    plsc.subcore_barrier()

    wid = cid * NTILES + sid
    ebase = wid * (E_TOT // 32)

    def deg_step(k, carry):
        b = ebase + k * CH
        pltpu.sync_copy(dst_hbm.at[pl.ds(b, CH)], idx_v)
        pltpu.sync_copy(ones_v, deg_sh.at[idx_v], add=True)
        return carry

    lax.fori_loop(0, NCW, deg_step, 0)

    @pl.when(cid == 0)
    def _():
        def cnt_step(k, carry):
            b = sid * (NSP // NTILES) + k * CH
            pltpu.sync_copy(batch_hbm.at[pl.ds(b, CH)], idx_v)
            pltpu.sync_copy(ones_v, cnt_sh.at[idx_v], add=True)
            return carry
        lax.fori_loop(0, NODE_NC, cnt_step, 0)

    def gather_half(t0_hbm, xg_hbm):
        def g_step(k, carry):
            b = sid * (NSP // NTILES) + k * CH
            pltpu.sync_copy(z_hbm.at[pl.ds(b, CH)], zidx_v)
            pltpu.async_copy(t0_hbm.at[zidx_v], rows_v, sem0).wait()
            pltpu.sync_copy(rows_v, xg_hbm.at[pl.ds(b, CH)])
            return carry
        lax.fori_loop(0, NODE_NC, g_step, 0)

    @pl.when(cid == 0)
    def _():
        gather_half(t0a_hbm, xga_hbm)

    @pl.when(cid == 1)
    def _():
        gather_half(t0b_hbm, xgb_hbm)

    plsc.subcore_barrier()

    def write_half(deg_hbm):
        for i in range(0, NROW_T, 1024):
            sz = min(1024, NROW_T - i)
            pltpu.sync_copy(deg_sh.at[pl.ds(base0 + i, sz)], obuf_v.at[pl.ds(0, sz)])
            pltpu.sync_copy(obuf_v.at[pl.ds(0, sz)], deg_hbm.at[pl.ds(base0 + i, sz)])

    @pl.when(cid == 0)
    def _():
        write_half(dega_hbm)

    @pl.when(cid == 1)
    def _():
        write_half(degb_hbm)

    @pl.when(jnp.logical_and(cid == 0, sid == 0))
    def _():
        pltpu.sync_copy(cnt_sh, obuf_v.at[pl.ds(0, 144)])
        pltpu.sync_copy(obuf_v.at[pl.ds(0, 144)], counts_hbm)


def _sc_deg(dst_pad, batch_pad, z_pad, t0a, t0b):
    return pl.kernel(
        _sc_deg_body,
        out_type=[
            jax.ShapeDtypeStruct((NP,), f32),
            jax.ShapeDtypeStruct((NP,), f32),
            jax.ShapeDtypeStruct((144,), f32),
            jax.ShapeDtypeStruct((NSP, HH), f32),
            jax.ShapeDtypeStruct((NSP, HH), f32),
        ],
        mesh=_mesh(),
        compiler_params=pltpu.CompilerParams(use_tc_tiling_on_sc=False, needs_layout_passes=False),
        scratch_types=[
            pltpu.VMEM_SHARED((NP,), f32),
            pltpu.VMEM_SHARED((144,), f32),
            pltpu.VMEM((CH,), f32),
            pltpu.VMEM((CH,), i32),
            pltpu.VMEM((CH,), i32),
            pltpu.VMEM((CH, HH), f32),
            pltpu.VMEM((1024,), f32),
            pltpu.SemaphoreType.DMA,
            pltpu.SemaphoreType.DMA,
        ],
    )(dst_pad, batch_pad, z_pad, t0a, t0b)


def _sc_edge_body(ya_hbm, yb_hbm, e_hbm, outa_hbm, outb_hbm,
                  acc_sh, e0, e1, e2, e3, r0, r1, r2, r3, zbuf_v,
                  sg0, sg1, sg2, sg3, ss0, ss1, ss2, ss3):
    cid = lax.axis_index("c")
    sid = lax.axis_index("s")
    eb = [e0, e1, e2, e3]
    rb = [r0, r1, r2, r3]
    sg = [sg0, sg1, sg2, sg3]
    ss = [ss0, ss1, ss2, ss3]

    zv = jnp.zeros((16,), f32)
    for i in range(1024):
        zbuf_v[i] = zv
    base0 = sid * NROW_T
    for i in range(0, NROW_T, 1024):
        sz = min(1024, NROW_T - i)
        pltpu.sync_copy(zbuf_v.at[pl.ds(0, sz)], acc_sh.at[pl.ds(base0 + i, sz)])
    plsc.subcore_barrier()

    tbase = sid * TE

    def run(y_hbm, out_hbm):
        def load_idx(b, k):
            pltpu.sync_copy(e_hbm.at[:, pl.ds(tbase + k * CH, CH)], eb[b])

        def start_g(b):
            return pltpu.async_copy(y_hbm.at[eb[b].at[0]], rb[b], sg[b])

        def start_s(b):
            return pltpu.async_copy(rb[b], acc_sh.at[eb[b].at[1]], ss[b], add=True)

        for j in (0, 1):
            load_idx(j, j)
            start_g(j)
        for ph in range(4):
            k = ph
            b = ph
            pltpu.make_async_copy(y_hbm.at[eb[b].at[0]], rb[b], sg[b]).wait()
            start_s(b)
            if k + 2 < 4:
                load_idx((k + 2) % 4, k + 2)
                start_g((k + 2) % 4)
            else:
                b2 = (k + 2) % 4
                pltpu.make_async_copy(rb[b2], acc_sh.at[eb[b2].at[1]], ss[b2]).wait()
                load_idx(b2, k + 2)
                start_g(b2)

        def group(g, carry):
            for ph in range(4):
                k = g * 4 + ph
                b = ph
                pltpu.make_async_copy(y_hbm.at[eb[b].at[0]], rb[b], sg[b]).wait()
                start_s(b)
                b2 = (ph + 2) % 4
                pltpu.make_async_copy(rb[b2], acc_sh.at[eb[b2].at[1]], ss[b2]).wait()
                pltpu.sync_copy(e_hbm.at[:, pl.ds(tbase + (k + 2) * CH, CH)], eb[b2])
                pltpu.async_copy(y_hbm.at[eb[b2].at[0]], rb[b2], sg[b2])
            return carry

        lax.fori_loop(1, NC // 4 - 1, group, 0)

        for ph in range(4):
            k = NC - 4 + ph
            b = ph
            pltpu.make_async_copy(y_hbm.at[eb[b].at[0]], rb[b], sg[b]).wait()
            start_s(b)
            if k + 2 < NC:
                b2 = (ph + 2) % 4
                pltpu.make_async_copy(rb[b2], acc_sh.at[eb[b2].at[1]], ss[b2]).wait()
                load_idx(b2, k + 2)
                start_g(b2)
        for ph in range(4):
            b = ph
            pltpu.make_async_copy(rb[b], acc_sh.at[eb[b].at[1]], ss[b]).wait()

        plsc.subcore_barrier()
        for i in range(0, NROW_T, 1024):
            sz = min(1024, NROW_T - i)
            pltpu.sync_copy(acc_sh.at[pl.ds(base0 + i, sz)], zbuf_v.at[pl.ds(0, sz)])
            pltpu.sync_copy(zbuf_v.at[pl.ds(0, sz)], out_hbm.at[pl.ds(base0 + i, sz)])

    @pl.when(cid == 0)
    def _():
        run(ya_hbm, outa_hbm)

    @pl.when(cid == 1)
    def _():
        run(yb_hbm, outb_hbm)


def _sc_edge(ya, yb, epad):
    return pl.kernel(
        _sc_edge_body,
        out_type=[
            jax.ShapeDtypeStruct((NP, HH), f32),
            jax.ShapeDtypeStruct((NP, HH), f32),
        ],
        mesh=_mesh(),
        compiler_params=pltpu.CompilerParams(use_tc_tiling_on_sc=False, needs_layout_passes=False),
        scratch_types=(
            [pltpu.VMEM_SHARED((NP, HH), f32)]
            + [pltpu.VMEM((2, CH), i32) for _ in range(4)]
            + [pltpu.VMEM((CH, HH), f32) for _ in range(4)]
            + [pltpu.VMEM((1024, HH), f32)]
            + [pltpu.SemaphoreType.DMA for _ in range(8)]
        ),
    )(ya, yb, epad)


def _sc_edge1_body(v_hbm, e_hbm, outa_hbm, outb_hbm,
                   acc_sh, e0, e1, e2, e3, r0, r1, r2, r3, zbuf_v,
                   sg0, sg1, sg2, sg3, ss0, ss1, ss2, ss3):
    cid = lax.axis_index("c")
    sid = lax.axis_index("s")
    eb = [e0, e1, e2, e3]
    rb = [r0, r1, r2, r3]
    sg = [sg0, sg1, sg2, sg3]
    ss = [ss0, ss1, ss2, ss3]

    _fill_vec(zbuf_v, 1024, 0.0, f32)
    base0 = sid * NROW_T
    for i in range(0, NROW_T, 1024):
        sz = min(1024, NROW_T - i)
        pltpu.sync_copy(zbuf_v.at[pl.ds(0, sz)], acc_sh.at[pl.ds(base0 + i, sz)])
    plsc.subcore_barrier()

    wid = cid * NTILES + sid
    tbase = wid * (E_TOT // 32)

    def load_idx(b, k):
        pltpu.sync_copy(e_hbm.at[:, pl.ds(tbase + k * CH, CH)], eb[b])

    def wait_g(b):
        pltpu.make_async_copy(v_hbm.at[eb[b].at[0]], rb[b], sg[b]).wait()

    def wait_s(b):
        pltpu.make_async_copy(rb[b], acc_sh.at[eb[b].at[1]], ss[b]).wait()

    for j in (0, 1):
        load_idx(j, j)
        pltpu.async_copy(v_hbm.at[eb[j].at[0]], rb[j], sg[j])
    for ph in range(4):
        k = ph
        b = ph
        wait_g(b)
        pltpu.async_copy(rb[b], acc_sh.at[eb[b].at[1]], ss[b], add=True)
        b2 = (k + 2) % 4
        if k + 2 < 4:
            load_idx(b2, k + 2)
            pltpu.async_copy(v_hbm.at[eb[b2].at[0]], rb[b2], sg[b2])
        else:
            wait_s(b2)
            load_idx(b2, k + 2)
            pltpu.async_copy(v_hbm.at[eb[b2].at[0]], rb[b2], sg[b2])

    def group(g, carry):
        for ph in range(4):
            k = g * 4 + ph
            b = ph
            wait_g(b)
            pltpu.async_copy(rb[b], acc_sh.at[eb[b].at[1]], ss[b], add=True)
            b2 = (ph + 2) % 4
            wait_s(b2)
            pltpu.sync_copy(e_hbm.at[:, pl.ds(tbase + (k + 2) * CH, CH)], eb[b2])
            pltpu.async_copy(v_hbm.at[eb[b2].at[0]], rb[b2], sg[b2])
        return carry

    lax.fori_loop(1, NCW // 4 - 1, group, 0)

    for ph in range(4):
        k = NCW - 4 + ph
        b = ph
        wait_g(b)
        pltpu.async_copy(rb[b], acc_sh.at[eb[b].at[1]], ss[b], add=True)
        if k + 2 < NCW:
            b2 = (ph + 2) % 4
            wait_s(b2)
            load_idx(b2, k + 2)
            pltpu.async_copy(v_hbm.at[eb[b2].at[0]], rb[b2], sg[b2])
    for ph in range(4):
        wait_s(ph)

    plsc.subcore_barrier()

    def write_half(out_hbm):
        for i in range(0, NROW_T, 1024):
            sz = min(1024, NROW_T - i)
            pltpu.sync_copy(acc_sh.at[pl.ds(base0 + i, sz)], zbuf_v.at[pl.ds(0, sz)])
            pltpu.sync_copy(zbuf_v.at[pl.ds(0, sz)], out_hbm.at[pl.ds(base0 + i, sz)])

    @pl.when(cid == 0)
    def _():
        write_half(outa_hbm)

    @pl.when(cid == 1)
    def _():
        write_half(outb_hbm)


def _sc_edge1(v, epad):
    return pl.kernel(
        _sc_edge1_body,
        out_type=[
            jax.ShapeDtypeStruct((NP,), f32),
            jax.ShapeDtypeStruct((NP,), f32),
        ],
        mesh=_mesh(),
        compiler_params=pltpu.CompilerParams(use_tc_tiling_on_sc=False, needs_layout_passes=False),
        scratch_types=(
            [pltpu.VMEM_SHARED((NP,), f32)]
            + [pltpu.VMEM((2, CH), i32) for _ in range(4)]
            + [pltpu.VMEM((CH,), f32) for _ in range(4)]
            + [pltpu.VMEM((1024,), f32)]
            + [pltpu.SemaphoreType.DMA for _ in range(8)]
        ),
    )(v, epad)


def _sc_pool_body(x4_hbm, batch_hbm, starts_hbm, keys_hbm,
                  neg_v, kv, bv, slot_v, starts_v, sem0):
    cid = lax.axis_index("c")
    sid = lax.axis_index("s")

    @pl.when(cid == 0)
    def _():
        _fill_vec(neg_v, 2080, NEG, f32)
        ibase = sid * (KEYSLEN // NTILES)
        for i in range(0, KEYSLEN // NTILES, 2080):
            pltpu.sync_copy(neg_v, keys_hbm.at[pl.ds(ibase + i, 2080)])
        pltpu.sync_copy(starts_hbm, starts_v)
        plsc.subcore_barrier()

        def step(k, carry):
            b = sid * (NSP // NTILES) + k * CH
            pltpu.sync_copy(batch_hbm.at[pl.ds(b, CH)], bv)
            pltpu.sync_copy(x4_hbm.at[pl.ds(b, CH)], kv)
            for j in range(8):
                b16 = bv[pl.ds(j * 16, 16)]
                s16 = plsc.load_gather(starts_v, [b16])
                ids16 = lax.iota(i32, 16) + (b + j * 16)
                pos = ids16 - s16
                ok = pos < C
                slot = jnp.where(ok, b16 * C + pos, 129 * C + lax.rem(pos, C))
                slot_v[pl.ds(j * 16, 16)] = slot
            pltpu.async_copy(kv, keys_hbm.at[slot_v], sem0).wait()
            return carry

        lax.fori_loop(0, NODE_NC, step, 0)


def _sc_pool(x4p, batch_pad, starts):
    return pl.kernel(
        _sc_pool_body,
        out_type=[jax.ShapeDtypeStruct((KEYSLEN,), f32)],
        mesh=_mesh(),
        compiler_params=pltpu.CompilerParams(use_tc_tiling_on_sc=False, needs_layout_passes=False),
        scratch_types=[
            pltpu.VMEM((2080,), f32),
            pltpu.VMEM((CH,), f32),
            pltpu.VMEM((CH,), i32),
            pltpu.VMEM((CH,), i32),
            pltpu.VMEM((144,), i32),
            pltpu.SemaphoreType.DMA,
        ],
    )(x4p, batch_pad, starts)


def _sc_sel_body(sel_hbm, x1_hbm, x2_hbm, x3_hbm, x4_hbm,
                 p1_hbm, p2_hbm, p3_hbm, p4_hbm,
                 sidx, rows_v, elems_v, sem0):
    cid = lax.axis_index("c")
    sid = lax.axis_index("s")
    nchunk = SEL // (NTILES * CH)

    def row_gather(x_hbm, p_hbm):
        for k in range(nchunk):
            b = (sid * nchunk + k) * CH
            pltpu.sync_copy(sel_hbm.at[pl.ds(b, CH)], sidx)
            pltpu.async_copy(x_hbm.at[sidx], rows_v, sem0).wait()
            pltpu.sync_copy(rows_v, p_hbm.at[pl.ds(b, CH)])

    @pl.when(cid == 0)
    def _():
        row_gather(x1_hbm, p1_hbm)
        row_gather(x2_hbm, p2_hbm)

    @pl.when(cid == 1)
    def _():
        row_gather(x3_hbm, p3_hbm)
        for k in range(nchunk):
            b = (sid * nchunk + k) * CH
            pltpu.sync_copy(sel_hbm.at[pl.ds(b, CH)], sidx)
            pltpu.async_copy(x4_hbm.at[sidx], elems_v, sem0).wait()
            pltpu.sync_copy(elems_v, p4_hbm.at[pl.ds(b, CH)])


def _sc_sel(sel_flat, x1, x2, x3, x4):
    return pl.kernel(
        _sc_sel_body,
        out_type=[
            jax.ShapeDtypeStruct((SEL, H), f32),
            jax.ShapeDtypeStruct((SEL, H), f32),
            jax.ShapeDtypeStruct((SEL, H), f32),
            jax.ShapeDtypeStruct((SEL,), f32),
        ],
        mesh=_mesh(),
        compiler_params=pltpu.CompilerParams(use_tc_tiling_on_sc=False, needs_layout_passes=False),
        scratch_types=[
            pltpu.VMEM((CH,), i32),
            pltpu.VMEM((CH, H), f32),
            pltpu.VMEM((CH,), f32),
            pltpu.SemaphoreType.DMA,
        ],
    )(sel_flat, x1, x2, x3, x4)


def _tc_pre_body(zt_ref, w0_ref, t0a_ref, t0b_ref):
    t0 = jnp.dot(zt_ref[...].astype(jnp.bfloat16), w0_ref[...].astype(jnp.bfloat16), preferred_element_type=f32)
    t0a_ref[...] = t0[:, :HH]
    t0b_ref[...] = t0[:, HH:]


def _tc_pre(z_table, W0):
    return pl.pallas_call(
        _tc_pre_body,
        out_shape=[
            jax.ShapeDtypeStruct((MAX_Z, HH), f32),
            jax.ShapeDtypeStruct((MAX_Z, HH), f32),
        ],
    )(z_table, W0)


BN = 2000


def _tc_b_body(dega_ref, degb_ref, xga_ref, xgb_ref,
               dinv_ref, y0a_ref, y0b_ref):
    deg = dega_ref[...] + degb_ref[...] + 1.0
    dinv = lax.rsqrt(deg)
    dinv_ref[...] = dinv
    y0a_ref[...] = xga_ref[...] * dinv
    y0b_ref[...] = xgb_ref[...] * dinv


def _tc_b(dega, degb, xga, xgb):
    grid = (N // BN,)
    col = pl.BlockSpec((BN, 1), lambda i: (i, 0))
    half = pl.BlockSpec((BN, HH), lambda i: (i, 0))
    return pl.pallas_call(
        _tc_b_body,
        grid=grid,
        in_specs=[col, col, half, half],
        out_specs=[col, half, half],
        out_shape=[
            jax.ShapeDtypeStruct((N, 1), f32),
            jax.ShapeDtypeStruct((N, HH), f32),
            jax.ShapeDtypeStruct((N, HH), f32),
        ],
    )(dega, degb, xga, xgb)


def _tc_starts_body(cnt_ref, starts_ref):
    c = cnt_ref[...].astype(i32)
    s = c
    zero = jnp.zeros((1, 144), i32)
    for sh in (1, 2, 4, 8, 16, 32, 64, 128):
        shifted = jnp.concatenate([zero[:, :sh], s[:, :144 - sh]], axis=1)
        s = s + shifted
    starts_ref[...] = s - c


def _tc_starts(counts):
    return pl.pallas_call(
        _tc_starts_body,
        out_shape=jax.ShapeDtypeStruct((1, 144), i32),
    )(counts.reshape(1, 144))


def _tc_layer_body(acca_ref, accb_ref, ya_ref, yb_ref, dinv_ref, w_ref, b_ref,
                   x_ref, yna_ref, ynb_ref):
    dinv = dinv_ref[...]
    fa = (acca_ref[...] + ya_ref[...]) * dinv + b_ref[0, :HH]
    fb = (accb_ref[...] + yb_ref[...]) * dinv + b_ref[0, HH:]
    x = jnp.tanh(jnp.concatenate([fa, fb], axis=1))
    x_ref[...] = x
    yn = jnp.dot(x.astype(jnp.bfloat16), w_ref[...].astype(jnp.bfloat16), preferred_element_type=f32) * dinv
    yna_ref[...] = yn[:, :HH]
    ynb_ref[...] = yn[:, HH:]


def _tc_layer(acca, accb, ya, yb, dinv, Wn, b_prev):
    grid = (N // BN,)
    col = pl.BlockSpec((BN, 1), lambda i: (i, 0))
    half = pl.BlockSpec((BN, HH), lambda i: (i, 0))
    full = pl.BlockSpec((BN, H), lambda i: (i, 0))
    wspec = pl.BlockSpec((H, H), lambda i: (0, 0))
    bspec = pl.BlockSpec((1, H), lambda i: (0, 0))
    return pl.pallas_call(
        _tc_layer_body,
        grid=grid,
        in_specs=[half, half, half, half, col, wspec, bspec],
        out_specs=[full, half, half],
        out_shape=[
            jax.ShapeDtypeStruct((N, H), f32),
            jax.ShapeDtypeStruct((N, HH), f32),
            jax.ShapeDtypeStruct((N, HH), f32),
        ],
    )(acca, accb, ya, yb, dinv, Wn, b_prev.reshape(1, H))


def _tc_layer3_body(acca_ref, accb_ref, ya_ref, yb_ref, dinv_ref, w_ref, b_ref,
                    x_ref, v_ref):
    dinv = dinv_ref[...]
    fa = (acca_ref[...] + ya_ref[...]) * dinv + b_ref[0, :HH]
    fb = (accb_ref[...] + yb_ref[...]) * dinv + b_ref[0, HH:]
    x = jnp.tanh(jnp.concatenate([fa, fb], axis=1))
    x_ref[...] = x
    v_ref[...] = jnp.dot(x.astype(jnp.bfloat16), w_ref[...].astype(jnp.bfloat16), preferred_element_type=f32) * dinv


def _tc_layer3(acca, accb, ya, yb, dinv, W3, b2):
    grid = (N // BN,)
    col = pl.BlockSpec((BN, 1), lambda i: (i, 0))
    half = pl.BlockSpec((BN, HH), lambda i: (i, 0))
    full = pl.BlockSpec((BN, H), lambda i: (i, 0))
    wspec = pl.BlockSpec((H, 1), lambda i: (0, 0))
    bspec = pl.BlockSpec((1, H), lambda i: (0, 0))
    return pl.pallas_call(
        _tc_layer3_body,
        grid=grid,
        in_specs=[half, half, half, half, col, wspec, bspec],
        out_specs=[full, col],
        out_shape=[
            jax.ShapeDtypeStruct((N, H), f32),
            jax.ShapeDtypeStruct((N, 1), f32),
        ],
    )(acca, accb, ya, yb, dinv, W3, b2.reshape(1, H))


def _tc_x4_body(acca_ref, accb_ref, v_ref, dinv_ref, b_ref, x4_ref):
    full = acca_ref[...] + accb_ref[...] + v_ref[...]
    x4_ref[...] = jnp.tanh(full * dinv_ref[...] + b_ref[0, 0])


def _tc_x4(acc4a, acc4b, v, dinv, b3):
    grid = (N // BN,)
    col = pl.BlockSpec((BN, 1), lambda i: (i, 0))
    bspec = pl.BlockSpec((1, 1), lambda i: (0, 0))
    return pl.pallas_call(
        _tc_x4_body,
        grid=grid,
        in_specs=[col, col, col, col, bspec],
        out_specs=col,
        out_shape=jax.ShapeDtypeStruct((N, 1), f32),
    )(acc4a.reshape(NP, 1), acc4b.reshape(NP, 1), v, dinv, b3.reshape(1, 1))


def _tc_topk_body(keys_ref, starts_ref, sel_ref, mask_ref):
    k = keys_ref[...]
    starts = starts_ref[...]
    idx2d = lax.broadcasted_iota(i32, (8, C), 1)
    BIG = jnp.int32(1 << 30)
    for t in range(K):
        m = jnp.max(k, axis=1, keepdims=True)
        slot = jnp.min(jnp.where(k >= m, idx2d, BIG), axis=1, keepdims=True)
        valid = m > -1.0e38
        node = jnp.where(valid, starts + slot, 0)
        sel_ref[:, t:t + 1] = node
        mask_ref[:, t:t + 1] = jnp.where(valid, 1.0, 0.0)
        k = jnp.where(idx2d == slot, NEG, k)
    zc = jnp.zeros((8, 1), i32)
    zf = jnp.zeros((8, 1), f32)
    for t in range(K, SELW):
        sel_ref[:, t:t + 1] = zc
        mask_ref[:, t:t + 1] = zf


def _tc_topk(keys2d, starts128):
    grid = (G // 8,)
    return pl.pallas_call(
        _tc_topk_body,
        grid=grid,
        in_specs=[
            pl.BlockSpec((8, C), lambda i: (i, 0)),
            pl.BlockSpec((8, 1), lambda i: (i, 0)),
        ],
        out_specs=[
            pl.BlockSpec((8, SELW), lambda i: (i, 0)),
            pl.BlockSpec((8, SELW), lambda i: (i, 0)),
        ],
        out_shape=[
            jax.ShapeDtypeStruct((G, SELW), i32),
            jax.ShapeDtypeStruct((G, SELW), f32),
        ],
    )(keys2d, starts128)


def _tc_head_body(p_ref, mask_ref, w1_ref, b1_ref, w2_ref, b2_ref,
                  l1_ref, l1b_ref, l2_ref, l2b_ref, out_ref):
    w1 = w1_ref[...]
    hs = []
    for t in range(K):
        blk = p_ref[:, t * TOTAL_LATENT:(t + 1) * TOTAL_LATENT]
        blk = blk * mask_ref[:, t:t + 1]
        h = jnp.dot(blk, w1, preferred_element_type=f32) + b1_ref[0, :]
        hs.append(jnp.maximum(h, 0.0))
    hm = [jnp.maximum(hs[2 * j], hs[2 * j + 1]) for j in range(K // 2)]
    fs = []
    for t in range(11):
        acc = jnp.zeros((G, 32), f32) + b2_ref[0, :]
        for dt in range(5):
            acc = acc + jnp.dot(hm[t + dt], w2_ref[dt], preferred_element_type=f32)
        fs.append(jnp.maximum(acc, 0.0))
    feat = jnp.concatenate(fs, axis=1)
    y = jnp.dot(feat, l1_ref[...], preferred_element_type=f32) + l1b_ref[0, :]
    y = jnp.maximum(y, 0.0)
    out_ref[...] = jnp.dot(y, l2_ref[...], preferred_element_type=f32) + l2b_ref[0, :]


def _tc_head(P, mask30, w1r, b1, w2r, b2, l1T, l1b, l2T, l2b):
    return pl.pallas_call(
        _tc_head_body,
        out_shape=jax.ShapeDtypeStruct((G, 1), f32),
    )(P, mask30, w1r, b1.reshape(1, 16), w2r, b2.reshape(1, 32),
      l1T, l1b.reshape(1, 128), l2T, l2b.reshape(1, 1))


def kernel(z, edge_index, batch, use_feature, embedding, z_table,
           W0, b0, W1, b1, W2, b2, W3, b3,
           conv1_w, conv1_b, conv2_w, conv2_b, lin1_w, lin1_b, lin2_w, lin2_b):
    src = edge_index[0]
    dst = edge_index[1]
    pad = E_TOT - E
    src_pad = jnp.concatenate([src, jnp.zeros((pad,), i32)])
    dst_pad = jnp.concatenate([dst, jnp.full((pad,), N, i32)])
    epad = jnp.stack([src_pad, dst_pad])
    batch_pad = jnp.concatenate([batch, jnp.full((NSP - N,), G, i32)])
    z_pad = jnp.concatenate([z, jnp.zeros((NSP - N,), i32)])

    t0a, t0b = _tc_pre(z_table, W0)
    dega, degb, counts, xga, xgb = _sc_deg(dst_pad, batch_pad, z_pad, t0a, t0b)
    dinv, y0a, y0b = _tc_b(dega.reshape(NP, 1), degb.reshape(NP, 1), xga, xgb)
    starts = _tc_starts(counts)

    acca, accb = _sc_edge(y0a, y0b, epad)
    x1, y1a, y1b = _tc_layer(acca, accb, y0a, y0b, dinv, W1, b0)
    acca, accb = _sc_edge(y1a, y1b, epad)
    x2, y2a, y2b = _tc_layer(acca, accb, y1a, y1b, dinv, W2, b1)
    acca, accb = _sc_edge(y2a, y2b, epad)
    x3, v = _tc_layer3(acca, accb, y2a, y2b, dinv, W3, b2)
    vflat = v.reshape(N)
    acc4a, acc4b = _sc_edge1(vflat, epad)
    x4 = _tc_x4(acc4a, acc4b, v, dinv, b3)

    x4p = jnp.concatenate([x4.reshape(N), jnp.zeros((NSP - N,), f32)])
    keys, = _sc_pool(x4p, batch_pad, starts.reshape(144))
    sel, maskv = _tc_topk(keys.reshape(GP, C)[:G], starts.reshape(144)[:G].reshape(G, 1))

    p1, p2, p3, p4 = _sc_sel(sel.reshape(SEL), x1, x2, x3, x4.reshape(N))
    P = jnp.concatenate([p1, p2, p3, p4[:, None]], axis=1)
    P = P.reshape(G, SELW, TOTAL_LATENT)[:, :K, :].reshape(G, K * TOTAL_LATENT)

    w1r = conv1_w[:, 0, :].T
    w2r = jnp.transpose(conv2_w, (2, 1, 0))
    l1p = lin1_w.reshape(128, 32, 11).transpose(0, 2, 1).reshape(128, DENSE_DIM)
    out = _tc_head(P, maskv[:, :K], w1r, conv1_b, w2r, conv2_b,
                   l1p.T, lin1_b, lin2_w.T, lin2_b)
    return out
